# Initial kernel scaffold; baseline (speedup 1.0000x reference)
import jax
import jax.numpy as jnp
from jax.experimental import pallas as pl


def kernel(x, edge_index, W_in, b_in, W_g0, b_g0, W_g1, b_g1, W_g2, b_g2, W_o1, b_o1, W_o2, b_o2):
    raise NotImplementedError("write your pallas kernel here")


# speedup vs baseline: 15.3403x; 15.3403x over previous
import functools

import jax
import jax.numpy as jnp
from jax import lax
from jax.experimental import pallas as pl
from jax.experimental.pallas import tpu as pltpu
from jax.experimental.pallas import tpu_sc as plsc

N = 10000
E = 320000
D_IN = 128
DH = 64

NC = 2
NS = 16
NW = NC * NS
CHUNK = 128
EPT = 10112
C = EPT // CHUNK
EPAD = NW * EPT - E

NP = 10112
RPT = NP // NS

NPD = 10240
RPTD = NPD // NS
WD = 16

FP = jnp.float32


def _mesh():
    return plsc.VectorSubcoreMesh(core_axis_name="c", subcore_axis_name="s")


def _make_agg():
    @functools.partial(
        pl.kernel,
        out_type=jax.ShapeDtypeStruct((NC, NP, DH), FP),
        mesh=_mesh(),
        scratch_types=[
            pltpu.VMEM((C, CHUNK), jnp.int32),
            pltpu.VMEM((C, CHUNK), jnp.int32),
            pltpu.VMEM((CHUNK, DH), FP),
            pltpu.VMEM_SHARED((NP, DH), FP),
            pltpu.SemaphoreType.DMA,
        ],
        compiler_params=pltpu.CompilerParams(use_tc_tiling_on_sc=False),
    )
    def agg(zeros_hbm, g_hbm, srcp_hbm, dstp_hbm, out_hbm,
            src_v, dst_v, rows_v, acc, sem):
        c = lax.axis_index("c")
        s = lax.axis_index("s")
        wid = c * NS + s
        row0 = s * RPT
        pltpu.sync_copy(zeros_hbm.at[pl.ds(row0, RPT)], acc.at[pl.ds(row0, RPT)])
        pltpu.sync_copy(srcp_hbm.at[wid], src_v)
        pltpu.sync_copy(dstp_hbm.at[wid], dst_v)
        plsc.subcore_barrier()

        def chunk(j, carry):
            pltpu.async_copy(g_hbm.at[src_v.at[j]], rows_v, sem).wait()
            pltpu.sync_copy(rows_v, acc.at[dst_v.at[j]], add=True)
            return carry

        lax.fori_loop(0, C, chunk, 0)
        plsc.subcore_barrier()
        pltpu.sync_copy(acc.at[pl.ds(row0, RPT)], out_hbm.at[c, pl.ds(row0, RPT)])

    return agg


def _make_deg():
    @functools.partial(
        pl.kernel,
        out_type=jax.ShapeDtypeStruct((NC, NPD, WD), FP),
        mesh=_mesh(),
        scratch_types=[
            pltpu.VMEM((C, CHUNK), jnp.int32),
            pltpu.VMEM((CHUNK, WD), FP),
            pltpu.VMEM_SHARED((NPD, WD), FP),
        ],
        compiler_params=pltpu.CompilerParams(use_tc_tiling_on_sc=False),
    )
    def deg(zeros_hbm, ones_hbm, dstp_hbm, out_hbm, dst_v, ones_v, acc):
        c = lax.axis_index("c")
        s = lax.axis_index("s")
        wid = c * NS + s
        row0 = s * RPTD
        pltpu.sync_copy(zeros_hbm.at[pl.ds(row0, RPTD)], acc.at[pl.ds(row0, RPTD)])
        pltpu.sync_copy(ones_hbm, ones_v)
        pltpu.sync_copy(dstp_hbm.at[wid], dst_v)
        plsc.subcore_barrier()

        def chunk(j, carry):
            pltpu.sync_copy(ones_v, acc.at[dst_v.at[j]], add=True)
            return carry

        lax.fori_loop(0, C, chunk, 0)
        plsc.subcore_barrier()
        pltpu.sync_copy(acc.at[pl.ds(row0, RPTD)], out_hbm.at[c, pl.ds(row0, RPTD)])

    return deg


def _mm(a, b):
    return lax.dot_general(a, b, (((1,), (0,)), ((), ())),
                           precision=lax.Precision.DEFAULT,
                           preferred_element_type=FP)


B = 2000


def _tc0_body(degp, x, w_in, b_in, w_g0, dis_o, g_o, d2h_o):
    deg = degp[:, 0] + degp[:, 1] + 1.0
    dis = lax.rsqrt(deg)[:, None]
    h0 = jnp.maximum(_mm(x[...], w_in[...]) + b_in[...], 0.0)
    hw = _mm(h0, w_g0[...])
    g = dis * hw
    dis_o[...] = dis
    g_o[...] = g
    d2h_o[...] = dis * g


def _tc_mid_body(aggp, d2h, dis, b_prev, w, g_o, d2h_o):
    a = aggp[0] + aggp[1]
    dis_v = dis[...]
    h = jnp.maximum(dis_v * a + d2h[...] + b_prev[...], 0.0)
    hw = _mm(h, w[...])
    g = dis_v * hw
    g_o[...] = g
    d2h_o[...] = dis_v * g


def _tc3_body(aggp, d2h, dis, b_g2, w_o1, b_o1, w_o2, b_o2, out):
    a = aggp[0] + aggp[1]
    dis_v = dis[...]
    h = jnp.maximum(dis_v * a + d2h[...] + b_g2[...], 0.0)
    t = jnp.maximum(_mm(h, w_o1[...]) + b_o1[...], 0.0)
    out[...] = _mm(t, w_o2[...]) + b_o2[...]


def _row_spec(d):
    return pl.BlockSpec((B, d), lambda i: (i, 0))


def _full_spec(r, d):
    return pl.BlockSpec((r, d), lambda i: (0, 0))


def _tc0(degp, x, w_in, b_in, w_g0):
    return pl.pallas_call(
        _tc0_body,
        grid=(N // B,),
        in_specs=[
            pl.BlockSpec((B, 2), lambda i: (i, 0)),
            _row_spec(D_IN),
            _full_spec(D_IN, DH),
            _full_spec(1, DH),
            _full_spec(DH, DH),
        ],
        out_specs=[_row_spec(1), _row_spec(DH), _row_spec(DH)],
        out_shape=[
            jax.ShapeDtypeStruct((N, 1), FP),
            jax.ShapeDtypeStruct((N, DH), FP),
            jax.ShapeDtypeStruct((N, DH), FP),
        ],
    )(degp, x, w_in, b_in, w_g0)


def _agg_spec():
    return pl.BlockSpec((2, B, DH), lambda i: (0, i, 0))


def _tc_mid(aggp, d2h, dis, b_prev, w):
    return pl.pallas_call(
        _tc_mid_body,
        grid=(N // B,),
        in_specs=[
            _agg_spec(),
            _row_spec(DH),
            _row_spec(1),
            _full_spec(1, DH),
            _full_spec(DH, DH),
        ],
        out_specs=[_row_spec(DH), _row_spec(DH)],
        out_shape=[
            jax.ShapeDtypeStruct((N, DH), FP),
            jax.ShapeDtypeStruct((N, DH), FP),
        ],
    )(aggp, d2h, dis, b_prev, w)


def _tc3(aggp, d2h, dis, b_g2, w_o1, b_o1, w_o2, b_o2):
    return pl.pallas_call(
        _tc3_body,
        grid=(N // B,),
        in_specs=[
            _agg_spec(),
            _row_spec(DH),
            _row_spec(1),
            _full_spec(1, DH),
            _full_spec(DH, DH // 2),
            _full_spec(1, DH // 2),
            _full_spec(DH // 2, 1),
            _full_spec(1, 1),
        ],
        out_specs=[_row_spec(1)],
        out_shape=[jax.ShapeDtypeStruct((N, 1), FP)],
    )(aggp, d2h, dis, b_g2, w_o1, b_o1, w_o2, b_o2)[0]


@jax.jit
def _run(x, edge_index, W_in, b_in, W_g0, b_g0, W_g1, b_g1, W_g2, b_g2,
         W_o1, b_o1, W_o2, b_o2):
    src = edge_index[0].astype(jnp.int32)
    dst = edge_index[1].astype(jnp.int32)
    srcp = jnp.concatenate([src, jnp.zeros((EPAD,), jnp.int32)]).reshape(NW, C, CHUNK)
    dstp = jnp.concatenate([dst, jnp.full((EPAD,), N, jnp.int32)]).reshape(NW, C, CHUNK)

    zeros_agg = jnp.zeros((NP, DH), FP)
    zeros_deg = jnp.zeros((NPD, WD), FP)
    ones_deg = jnp.ones((CHUNK, WD), FP)

    agg = _make_agg()
    degp = _make_deg()(zeros_deg, ones_deg, dstp)
    degp = degp[:, :N, 0].T

    dis, g0, d2h0 = _tc0(degp, x, W_in, b_in.reshape(1, DH), W_g0)

    a0 = agg(zeros_agg, g0, srcp, dstp)[:, :N, :]
    g1, d2h1 = _tc_mid(a0, d2h0, dis, b_g0.reshape(1, DH), W_g1)

    a1 = agg(zeros_agg, g1, srcp, dstp)[:, :N, :]
    g2, d2h2 = _tc_mid(a1, d2h1, dis, b_g1.reshape(1, DH), W_g2)

    a2 = agg(zeros_agg, g2, srcp, dstp)[:, :N, :]
    out = _tc3(a2, d2h2, dis, b_g2.reshape(1, DH), W_o1,
               b_o1.reshape(1, DH // 2), W_o2, b_o2.reshape(1, 1))
    return out


def kernel(x, edge_index, W_in, b_in, W_g0, b_g0, W_g1, b_g1, W_g2, b_g2,
           W_o1, b_o1, W_o2, b_o2):
    return _run(x, edge_index, W_in, b_in, W_g0, b_g0, W_g1, b_g1, W_g2, b_g2,
                W_o1, b_o1, W_o2, b_o2)
